# baseline (speedup 1.0000x reference)
import jax
import jax.numpy as jnp
from jax.experimental import pallas as pl
from jax.experimental.pallas import tpu as pltpu

B, T, N, F, H = 4, 8, 256, 16, 16
_EPS = 1e-5
_f32 = jnp.float32

_RW1, _RW2, _RF1, _RF2 = 0, 16, 32, 192
_RIH1, _RHH1, _RIH2, _RHH2, _RBIAS = 208, 272, 336, 400, 464


def _dot(a, b, ca, cb):
    return jax.lax.dot_general(
        a, b, (((ca,), (cb,)), ((), ())), preferred_element_type=_f32)


def _body(A_ref, X_ref, Wp_ref, out_ref, hbuf, fcacc):
    cnt = float(B * T * N)
    W1 = Wp_ref[_RW1:_RW1 + F, 0:H]
    W2 = Wp_ref[_RW2:_RW2 + H, 0:H]
    b1c = Wp_ref[_RBIAS:_RBIAS + H, 2:3]
    b2c = Wp_ref[_RBIAS:_RBIAS + H, 5:6]

    def gcn_t(Ab, xwT, bias):
        deg = jnp.sum(Ab, axis=0, keepdims=True) + 1.0
        dinv = jax.lax.rsqrt(deg)
        vT = _dot(dinv * xwT, Ab, 1, 0)
        z = dinv * vT + (dinv * dinv) * xwT + bias
        return jnp.maximum(z, 0.0)

    s1 = jnp.zeros((H, 1), _f32)
    q1 = jnp.zeros((H, 1), _f32)
    for b in range(B):
        fcsum = jnp.zeros((H, N), _f32)
        xwT_all = _dot(W1, X_ref[b].reshape(T * N, F), 0, 1)
        for t in range(T):
            r = gcn_t(A_ref[b, t], xwT_all[:, t * N:(t + 1) * N], b1c)
            hbuf[t, 0:H, b * N:(b + 1) * N] = r
            s1 = s1 + jnp.sum(r, axis=1, keepdims=True)
            q1 = q1 + jnp.sum(r * r, axis=1, keepdims=True)
            fcsum = fcsum + _dot(
                Wp_ref[_RF1 + 2 * H + t * F:_RF1 + 2 * H + (t + 1) * F, 0:H],
                X_ref[b, t], 0, 1)
        fcacc[:, b * N:(b + 1) * N] = fcsum

    mean1 = s1 / cnt
    var1 = q1 / cnt - mean1 * mean1
    sc1 = Wp_ref[_RBIAS:_RBIAS + H, 3:4] * jax.lax.rsqrt(var1 + _EPS)
    sh1 = Wp_ref[_RBIAS:_RBIAS + H, 4:5] - mean1 * sc1

    s2 = jnp.zeros((H, 1), _f32)
    q2 = jnp.zeros((H, 1), _f32)
    for b in range(B):
        for t in range(T):
            cols = slice(b * N, (b + 1) * N)
            h1n = hbuf[t, 0:H, cols] * sc1 + sh1
            hbuf[t, 0:H, cols] = h1n
            xw2T = _dot(W2, h1n, 0, 0)
            r2 = gcn_t(A_ref[b, t], xw2T, b2c)
            hbuf[t, H:2 * H, cols] = r2
            s2 = s2 + jnp.sum(r2, axis=1, keepdims=True)
            q2 = q2 + jnp.sum(r2 * r2, axis=1, keepdims=True)

    mean2 = s2 / cnt
    var2 = q2 / cnt - mean2 * mean2
    sc2 = Wp_ref[_RBIAS:_RBIAS + H, 6:7] * jax.lax.rsqrt(var2 + _EPS)
    sh2 = Wp_ref[_RBIAS:_RBIAS + H, 7:8] - mean2 * sc2

    BN_ = B * N
    h1 = jnp.zeros((H, BN_), _f32)
    c1 = jnp.zeros((H, BN_), _f32)
    h2 = jnp.zeros((H, BN_), _f32)
    c2 = jnp.zeros((H, BN_), _f32)
    Wih1 = Wp_ref[_RIH1:_RIH1 + 4 * H, 0:2 * H]
    Whh1 = Wp_ref[_RHH1:_RHH1 + 4 * H, 0:H]
    Wih2 = Wp_ref[_RIH2:_RIH2 + 4 * H, 0:H]
    Whh2 = Wp_ref[_RHH2:_RHH2 + 4 * H, 0:H]
    lb1 = Wp_ref[_RBIAS:_RBIAS + 4 * H, 0:1]
    lb2 = Wp_ref[_RBIAS:_RBIAS + 4 * H, 1:2]

    def gates_act(g, c):
        i_ = jax.nn.sigmoid(g[0:H])
        f_ = jax.nn.sigmoid(g[H:2 * H])
        gg = jnp.tanh(g[2 * H:3 * H])
        o_ = jax.nn.sigmoid(g[3 * H:4 * H])
        c = f_ * c + i_ * gg
        return o_ * jnp.tanh(c), c

    for tt in range(T):
        xt = jnp.concatenate(
            [hbuf[tt, 0:H, :], hbuf[tt, H:2 * H, :] * sc2 + sh2], axis=0)
        g1v = _dot(Wih1, xt, 1, 0) + _dot(Whh1, h1, 1, 0) + lb1
        h1, c1 = gates_act(g1v, c1)
        g2v = _dot(Wih2, h1, 1, 0) + _dot(Whh2, h2, 1, 0) + lb2
        h2, c2 = gates_act(g2v, c2)

    pre = (fcacc[...] + _dot(Wp_ref[_RF1:_RF1 + H, 0:H], h1, 0, 0)
           + _dot(Wp_ref[_RF1 + H:_RF1 + 2 * H, 0:H], h2, 0, 0)
           + Wp_ref[_RBIAS:_RBIAS + H, 8:9])
    y1 = jnp.maximum(pre, 0.0)
    out_ref[...] = jnp.maximum(
        _dot(Wp_ref[_RF2:_RF2 + H, 0:1], y1, 0, 0)
        + Wp_ref[_RBIAS:_RBIAS + 1, 9:10], 0.0)


def kernel(X, y, A, W1, b1, g1, be1, W2, b2, g2, be2,
           Wih1, Whh1, bih1, bhh1, Wih2, Whh2, bih2, bhh2,
           Wf1, bf1, Wf2, bf2):
    padc = lambda m: jnp.pad(m.astype(_f32), ((0, 0), (0, 32 - m.shape[1])))
    colp = lambda v: jnp.pad(v.astype(_f32).reshape(-1, 1),
                             ((0, 64 - v.shape[0]), (0, 0)))
    bias_block = jnp.concatenate(
        [(bih1 + bhh1).astype(_f32).reshape(-1, 1),
         (bih2 + bhh2).astype(_f32).reshape(-1, 1),
         colp(b1), colp(g1), colp(be1), colp(b2), colp(g2), colp(be2),
         colp(bf1), colp(bf2)], axis=1)
    Wpack = jnp.concatenate(
        [padc(W1), padc(W2), padc(Wf1), padc(Wf2), padc(Wih1), padc(Whh1),
         padc(Wih2), padc(Whh2), jnp.pad(bias_block, ((0, 0), (0, 22)))],
        axis=0)

    full = lambda arr: pl.BlockSpec(arr.shape, lambda: (0,) * arr.ndim)
    operands = [A, X.astype(_f32), Wpack]

    out = pl.pallas_call(
        _body,
        in_specs=[full(op) for op in operands],
        out_specs=pl.BlockSpec((1, B * N), lambda: (0, 0)),
        out_shape=jax.ShapeDtypeStruct((1, B * N), _f32),
        scratch_shapes=[
            pltpu.VMEM((T, 2 * H, B * N), _f32),
            pltpu.VMEM((H, B * N), _f32),
        ],
    )(*operands)
    return out.reshape(B, 1, N, 1)


# speedup vs baseline: 1.2040x; 1.2040x over previous
import jax
import jax.numpy as jnp
from jax.experimental import pallas as pl
from jax.experimental.pallas import tpu as pltpu

B, T, N, F, H = 4, 8, 256, 16, 16
_EPS = 1e-5
_f32 = jnp.float32

_RW1, _RW2, _RF1, _RHH1, _RIH2, _RHH2 = 0, 16, 32, 192, 256, 320
_RB = 384
_RLB = 392
_RBF2 = 408


def _dot(a, b, ca, cb):
    return jax.lax.dot_general(
        a, b, (((ca,), (cb,)), ((), ())), preferred_element_type=_f32)


def _body(A_ref, X_ref, Wp_ref, Wih1_ref, out_ref, hbuf, fcacc):
    cnt = float(B * T * N)
    W1 = Wp_ref[_RW1:_RW1 + F, :]
    W2 = Wp_ref[_RW2:_RW2 + H, :]
    bias_t = jnp.swapaxes(Wp_ref[_RB:_RB + 24, :], 0, 1)
    colv = lambda j: bias_t[:, j:j + 1]
    b1c = colv(0)
    b2c = colv(3)

    def gcn_t(Ab, xwT, bias):
        deg = jnp.sum(Ab, axis=0, keepdims=True) + 1.0
        dinv = jax.lax.rsqrt(deg)
        vT = _dot(dinv * xwT, Ab, 1, 0)
        z = dinv * vT + (dinv * dinv) * xwT + bias
        return jnp.maximum(z, 0.0)

    s1 = jnp.zeros((H, 1), _f32)
    q1 = jnp.zeros((H, 1), _f32)
    for b in range(B):
        fcsum = jnp.zeros((H, N), _f32)
        xwT_all = _dot(W1, X_ref[b].reshape(T * N, F), 0, 1)
        for t in range(T):
            r = gcn_t(A_ref[b, t], xwT_all[:, t * N:(t + 1) * N], b1c)
            hbuf[t, 0:H, b * N:(b + 1) * N] = r
            s1 = s1 + jnp.sum(r, axis=1, keepdims=True)
            q1 = q1 + jnp.sum(r * r, axis=1, keepdims=True)
            fcsum = fcsum + _dot(
                Wp_ref[_RF1 + 2 * H + t * F:_RF1 + 2 * H + (t + 1) * F, :],
                X_ref[b, t], 0, 1)
        fcacc[:, b * N:(b + 1) * N] = fcsum

    mean1 = s1 / cnt
    var1 = q1 / cnt - mean1 * mean1
    sc1 = colv(1) * jax.lax.rsqrt(var1 + _EPS)
    sh1 = colv(2) - mean1 * sc1

    s2 = jnp.zeros((H, 1), _f32)
    q2 = jnp.zeros((H, 1), _f32)
    for b in range(B):
        for t in range(T):
            cols = slice(b * N, (b + 1) * N)
            h1n = hbuf[t, 0:H, cols] * sc1 + sh1
            hbuf[t, 0:H, cols] = h1n
            xw2T = _dot(W2, h1n, 0, 0)
            r2 = gcn_t(A_ref[b, t], xw2T, b2c)
            hbuf[t, H:2 * H, cols] = r2
            s2 = s2 + jnp.sum(r2, axis=1, keepdims=True)
            q2 = q2 + jnp.sum(r2 * r2, axis=1, keepdims=True)

    mean2 = s2 / cnt
    var2 = q2 / cnt - mean2 * mean2
    sc2 = colv(4) * jax.lax.rsqrt(var2 + _EPS)
    sh2 = colv(5) - mean2 * sc2

    BN_ = B * N
    h1 = jnp.zeros((H, BN_), _f32)
    c1 = jnp.zeros((H, BN_), _f32)
    h2 = jnp.zeros((H, BN_), _f32)
    c2 = jnp.zeros((H, BN_), _f32)
    Wih1 = Wih1_ref[...]
    Whh1 = Wp_ref[_RHH1:_RHH1 + 4 * H, :]
    Wih2 = Wp_ref[_RIH2:_RIH2 + 4 * H, :]
    Whh2 = Wp_ref[_RHH2:_RHH2 + 4 * H, :]
    lbs1 = [bias_t[:, 8 + j:9 + j] + bias_t[:, 12 + j:13 + j]
            for j in range(4)]
    lbs2 = [bias_t[:, 16 + j:17 + j] + bias_t[:, 20 + j:21 + j]
            for j in range(4)]

    def gates_act(g, c, lbs):
        i_ = jax.nn.sigmoid(g[0:H] + lbs[0])
        f_ = jax.nn.sigmoid(g[H:2 * H] + lbs[1])
        gg = jnp.tanh(g[2 * H:3 * H] + lbs[2])
        o_ = jax.nn.sigmoid(g[3 * H:4 * H] + lbs[3])
        c = f_ * c + i_ * gg
        return o_ * jnp.tanh(c), c

    for tt in range(T):
        xt = jnp.concatenate(
            [hbuf[tt, 0:H, :], hbuf[tt, H:2 * H, :] * sc2 + sh2], axis=0)
        g1v = _dot(Wih1, xt, 1, 0) + _dot(Whh1, h1, 1, 0)
        h1, c1 = gates_act(g1v, c1, lbs1)
        g2v = _dot(Wih2, h1, 1, 0) + _dot(Whh2, h2, 1, 0)
        h2, c2 = gates_act(g2v, c2, lbs2)

    pre = (fcacc[...] + _dot(Wp_ref[_RF1:_RF1 + H, :], h1, 0, 0)
           + _dot(Wp_ref[_RF1 + H:_RF1 + 2 * H, :], h2, 0, 0)
           + colv(6))
    y1 = jnp.maximum(pre, 0.0)
    out_ref[...] = jnp.maximum(
        _dot(colv(7), y1, 0, 0) + Wp_ref[_RBF2:_RBF2 + 1, 0:1], 0.0)


def kernel(X, y, A, W1, b1, g1, be1, W2, b2, g2, be2,
           Wih1, Whh1, bih1, bhh1, Wih2, Whh2, bih2, bhh2,
           Wf1, bf1, Wf2, bf2):
    bias1d = jnp.concatenate([
        b1, g1, be1, b2, g2, be2, bf1, Wf2.reshape(-1),
        bih1, bhh1, bih2, bhh2, bf2])
    bias2d = jnp.pad(bias1d, (0, 512 - bias1d.shape[0])).reshape(32, 16)
    Wpack = jnp.concatenate(
        [W1, W2, Wf1, Whh1, Wih2, Whh2, bias2d], axis=0)

    full = lambda arr: pl.BlockSpec(arr.shape, lambda: (0,) * arr.ndim)
    operands = [A, X, Wpack, Wih1]

    out = pl.pallas_call(
        _body,
        in_specs=[full(op) for op in operands],
        out_specs=pl.BlockSpec((1, B * N), lambda: (0, 0)),
        out_shape=jax.ShapeDtypeStruct((1, B * N), _f32),
        scratch_shapes=[
            pltpu.VMEM((T, 2 * H, B * N), _f32),
            pltpu.VMEM((H, B * N), _f32),
        ],
    )(*operands)
    return out.reshape(B, 1, N, 1)
